# Initial kernel scaffold; baseline (speedup 1.0000x reference)
import jax
import jax.numpy as jnp
from jax.experimental import pallas as pl


def kernel(pos, vel, F, C, Jp):
    raise NotImplementedError("write your pallas kernel here")


# speedup vs baseline: 84.2394x; 84.2394x over previous
import functools

import jax
import jax.numpy as jnp
from jax import lax
from jax.experimental import pallas as pl
from jax.experimental.pallas import tpu as pltpu
from jax.experimental.pallas import tpu_sc as plsc

RES = 256
DT = 1e-4
E_MOD = 10000.0
NU = 0.2
RHO = 1.0
MU0 = E_MOD / (2.0 * (1.0 + NU))
LAM0 = E_MOD * NU / ((1.0 + NU) * (1.0 - 2.0 * NU))
DX = 1.0 / RES
VOL = DX * DX * 0.5
RV = RHO * VOL
KS = -DT * VOL * (4.0 / DX ** 2)
N_P = 524288
N_TILES = 32
P_TILE = N_P // N_TILES
CH = 1024
N_CHUNK = P_TILE // CH
GROUPS = CH // 16
HALF = (RES * RES) // 2

_mesh = plsc.VectorSubcoreMesh(core_axis_name="c", subcore_axis_name="s")


@functools.partial(
    pl.kernel,
    out_type=jax.ShapeDtypeStruct((6, N_TILES, 1, HALF), jnp.float32),
    mesh=_mesh,
    compiler_params=pltpu.CompilerParams(needs_layout_passes=False),
    scratch_types=[
        pltpu.VMEM((CH * 2,), jnp.float32),
        pltpu.VMEM((CH * 2,), jnp.float32),
        pltpu.VMEM((CH * 4,), jnp.float32),
        pltpu.VMEM((CH * 4,), jnp.float32),
        pltpu.VMEM((HALF,), jnp.float32),
        pltpu.VMEM((HALF,), jnp.float32),
        pltpu.VMEM((HALF,), jnp.float32),
    ],
)
def _p2g_sc(pos_hbm, vel_hbm, f_hbm, c_hbm, out_hbm,
            pos_v, vel_v, f_v, c_v, mass_v, mx_v, my_v):
    wid = lax.axis_index("s") * 2 + lax.axis_index("c")
    pbase = wid * P_TILE
    iota = lax.broadcasted_iota(jnp.int32, (16,), 0)
    iota2 = iota * 2
    iota4 = iota * 4
    zf = jnp.zeros((16,), jnp.float32)

    for half in range(2):
        def zero_body(i, carry):
            sl = pl.ds(i * 16, 16)
            mass_v[sl] = zf
            mx_v[sl] = zf
            my_v[sl] = zf
            return carry
        lax.fori_loop(0, HALF // 16, zero_body, 0)

        def chunk_body(k, carry):
            src2 = pl.ds((pbase + k * CH) * 2, CH * 2)
            src4 = pl.ds((pbase + k * CH) * 4, CH * 4)
            pltpu.sync_copy(pos_hbm.at[src2], pos_v)
            pltpu.sync_copy(vel_hbm.at[src2], vel_v)
            pltpu.sync_copy(f_hbm.at[src4], f_v)
            pltpu.sync_copy(c_hbm.at[src4], c_v)

            def group_body(g, gcarry):
                r2 = g * 32 + iota2
                r2b = r2 + 1
                r4 = g * 64 + iota4
                px = plsc.load_gather(pos_v, [r2])
                py = plsc.load_gather(pos_v, [r2b])
                vx = plsc.load_gather(vel_v, [r2])
                vy = plsc.load_gather(vel_v, [r2b])
                f00 = plsc.load_gather(f_v, [r4])
                f01 = plsc.load_gather(f_v, [r4 + 1])
                f10 = plsc.load_gather(f_v, [r4 + 2])
                f11 = plsc.load_gather(f_v, [r4 + 3])
                c00 = plsc.load_gather(c_v, [r4])
                c01 = plsc.load_gather(c_v, [r4 + 1])
                c10 = plsc.load_gather(c_v, [r4 + 2])
                c11 = plsc.load_gather(c_v, [r4 + 3])

                det = f00 * f11 - f01 * f10
                rdet = 1.0 / det
                j_cl = jnp.maximum(det, 1e-8)
                nrdet = -rdet
                ft00 = f11 * rdet
                ft01 = f10 * nrdet
                ft10 = f01 * nrdet
                ft11 = f00 * rdet
                b = LAM0 * (j_cl - 1.0) - MU0
                p00 = MU0 * f00 + b * ft00
                p01 = MU0 * f01 + b * ft01
                p10 = MU0 * f10 + b * ft10
                p11 = MU0 * f11 + b * ft11
                a00 = KS * (p00 * f00 + p01 * f01) + RV * c00
                a01 = KS * (p00 * f10 + p01 * f11) + RV * c01
                a10 = KS * (p10 * f00 + p11 * f01) + RV * c10
                a11 = KS * (p10 * f10 + p11 * f11) + RV * c11

                xpx = px * float(RES) - 0.5
                xpy = py * float(RES) - 0.5
                bx = xpx.astype(jnp.int32)
                by = xpy.astype(jnp.int32)
                fxx = xpx - bx.astype(jnp.float32)
                fxy = xpy - by.astype(jnp.float32)

                tx0 = 1.5 - fxx
                tx1 = fxx - 1.0
                tx2 = fxx - 0.5
                ty0 = 1.5 - fxy
                ty1 = fxy - 1.0
                ty2 = fxy - 0.5
                wx = (0.5 * (tx0 * tx0),
                      0.75 - tx1 * tx1,
                      0.5 * (tx2 * tx2))
                wy = (0.5 * (ty0 * ty0),
                      0.75 - ty1 * ty1,
                      0.5 * (ty2 * ty2))
                wxm = tuple(w * RV for w in wx)

                mvx = RV * vx
                mvy = RV * vy
                dpx = tuple((float(i) - fxx) * DX for i in range(3))
                dpy = tuple((float(j) - fxy) * DX for j in range(3))
                xu = tuple(mvx + a00 * d for d in dpx)
                yv = tuple(mvy + a10 * d for d in dpx)
                v_j = tuple(a01 * d for d in dpy)
                s_j = tuple(a11 * d for d in dpy)

                nxs = tuple(
                    jnp.clip(bx + i, 0, RES - 1) << 8 for i in range(3))
                nys = tuple(
                    jnp.clip(by + j, 0, RES - 1) for j in range(3))

                for i in range(3):
                    for j in range(3):
                        flat = nxs[i] + nys[j]
                        ww = wx[i] * wy[j]
                        mass_c = wxm[i] * wy[j]
                        momx_c = ww * (xu[i] + v_j[j])
                        momy_c = ww * (yv[i] + s_j[j])
                        if half == 0:
                            idx = flat
                            m = flat < HALF
                        else:
                            idx = flat - HALF
                            m = flat >= HALF
                        plsc.addupdate_scatter(mass_v, [idx], mass_c, mask=m)
                        plsc.addupdate_scatter(mx_v, [idx], momx_c, mask=m)
                        plsc.addupdate_scatter(my_v, [idx], momy_c, mask=m)
                return gcarry

            lax.fori_loop(0, GROUPS, group_body, 0)
            return carry

        lax.fori_loop(0, N_CHUNK, chunk_body, 0)

        pltpu.sync_copy(mass_v, out_hbm.at[3 * half + 0, wid, 0])
        pltpu.sync_copy(mx_v, out_hbm.at[3 * half + 1, wid, 0])
        pltpu.sync_copy(my_v, out_hbm.at[3 * half + 2, wid, 0])


def _red_body(x_ref, o_ref):
    o_ref[...] = jnp.sum(x_ref[...], axis=1)


_reduce_tc = pl.pallas_call(
    _red_body,
    grid=(6,),
    in_specs=[pl.BlockSpec((1, N_TILES, 1, HALF), lambda k: (k, 0, 0, 0))],
    out_specs=pl.BlockSpec((1, 1, HALF), lambda k: (k, 0, 0)),
    out_shape=jax.ShapeDtypeStruct((6, 1, HALF), jnp.float32),
)


def kernel(pos, vel, F, C, Jp):
    del Jp
    parts = _p2g_sc(pos.reshape(-1), vel.reshape(-1),
                    F.reshape(-1), C.reshape(-1))
    red = _reduce_tc(parts).reshape(2, 3, HALF)
    grid_mass = jnp.concatenate([red[0, 0], red[1, 0]])
    momx = jnp.concatenate([red[0, 1], red[1, 1]])
    momy = jnp.concatenate([red[0, 2], red[1, 2]])
    grid_mom = jnp.stack([momx, momy], axis=-1)
    return grid_mom, grid_mass


# speedup vs baseline: 723.8782x; 8.5931x over previous
import functools

import jax
import jax.numpy as jnp
from jax import lax
from jax.experimental import pallas as pl
from jax.experimental.pallas import tpu as pltpu
from jax.experimental.pallas import tpu_sc as plsc

RES = 256
DT = 1e-4
E_MOD = 10000.0
NU = 0.2
RHO = 1.0
MU0 = E_MOD / (2.0 * (1.0 + NU))
LAM0 = E_MOD * NU / ((1.0 + NU) * (1.0 - 2.0 * NU))
DX = 1.0 / RES
VOL = DX * DX * 0.5
RV = RHO * VOL
KS = -DT * VOL * (4.0 / DX ** 2)
N_P = 524288
N_TILES = 32
P_TILE = N_P // N_TILES
CH = 1024
N_CHUNK = P_TILE // CH
GROUPS = CH // 16
HALF = (RES * RES) // 2

_mesh = plsc.VectorSubcoreMesh(core_axis_name="c", subcore_axis_name="s")


@functools.partial(
    pl.kernel,
    out_type=jax.ShapeDtypeStruct((6, N_TILES, 1, HALF), jnp.float32),
    mesh=_mesh,
    compiler_params=pltpu.CompilerParams(needs_layout_passes=False),
    scratch_types=[
        [pltpu.VMEM((CH,), jnp.float32) for _ in range(12)],
        pltpu.VMEM((HALF,), jnp.float32),
        pltpu.VMEM((HALF,), jnp.float32),
        pltpu.VMEM((HALF,), jnp.float32),
    ],
)
def _p2g_sc(px_h, py_h, vx_h, vy_h, f00_h, f01_h, f10_h, f11_h,
            c00_h, c01_h, c10_h, c11_h, out_hbm,
            comp_v, mass_v, mx_v, my_v):
    inputs_h = (px_h, py_h, vx_h, vy_h, f00_h, f01_h, f10_h, f11_h,
                c00_h, c01_h, c10_h, c11_h)
    wid = lax.axis_index("s") * 2 + lax.axis_index("c")
    pbase = wid * P_TILE
    zf = jnp.zeros((16,), jnp.float32)

    for half in range(2):
        def zero_body(i, carry):
            sl = pl.ds(i * 16, 16)
            mass_v[sl] = zf
            mx_v[sl] = zf
            my_v[sl] = zf
            return carry
        lax.fori_loop(0, HALF // 16, zero_body, 0)

        def chunk_body(k, carry):
            src = pl.ds(pbase + k * CH, CH)
            for h_ref, v_ref in zip(inputs_h, comp_v):
                pltpu.sync_copy(h_ref.at[src], v_ref)

            def group_body(g, gcarry):
                sl = pl.ds(g * 16, 16)
                px = comp_v[0][sl]
                py = comp_v[1][sl]
                vx = comp_v[2][sl]
                vy = comp_v[3][sl]
                f00 = comp_v[4][sl]
                f01 = comp_v[5][sl]
                f10 = comp_v[6][sl]
                f11 = comp_v[7][sl]
                c00 = comp_v[8][sl]
                c01 = comp_v[9][sl]
                c10 = comp_v[10][sl]
                c11 = comp_v[11][sl]

                det = f00 * f11 - f01 * f10
                rdet = 1.0 / det
                j_cl = jnp.maximum(det, 1e-8)
                nrdet = -rdet
                ft00 = f11 * rdet
                ft01 = f10 * nrdet
                ft10 = f01 * nrdet
                ft11 = f00 * rdet
                b = LAM0 * (j_cl - 1.0) - MU0
                p00 = MU0 * f00 + b * ft00
                p01 = MU0 * f01 + b * ft01
                p10 = MU0 * f10 + b * ft10
                p11 = MU0 * f11 + b * ft11
                a00 = KS * (p00 * f00 + p01 * f01) + RV * c00
                a01 = KS * (p00 * f10 + p01 * f11) + RV * c01
                a10 = KS * (p10 * f00 + p11 * f01) + RV * c10
                a11 = KS * (p10 * f10 + p11 * f11) + RV * c11

                xpx = px * float(RES) - 0.5
                xpy = py * float(RES) - 0.5
                bx = xpx.astype(jnp.int32)
                by = xpy.astype(jnp.int32)
                fxx = xpx - bx.astype(jnp.float32)
                fxy = xpy - by.astype(jnp.float32)

                tx0 = 1.5 - fxx
                tx1 = fxx - 1.0
                tx2 = fxx - 0.5
                ty0 = 1.5 - fxy
                ty1 = fxy - 1.0
                ty2 = fxy - 0.5
                wx = (0.5 * (tx0 * tx0),
                      0.75 - tx1 * tx1,
                      0.5 * (tx2 * tx2))
                wy = (0.5 * (ty0 * ty0),
                      0.75 - ty1 * ty1,
                      0.5 * (ty2 * ty2))
                wxm = tuple(w * RV for w in wx)

                mvx = RV * vx
                mvy = RV * vy
                dpx = tuple((float(i) - fxx) * DX for i in range(3))
                dpy = tuple((float(j) - fxy) * DX for j in range(3))
                xu = tuple(mvx + a00 * d for d in dpx)
                yv = tuple(mvy + a10 * d for d in dpx)
                v_j = tuple(a01 * d for d in dpy)
                s_j = tuple(a11 * d for d in dpy)

                nxs = tuple(
                    jnp.clip(bx + i, 0, RES - 1) << 8 for i in range(3))
                nys = tuple(
                    jnp.clip(by + j, 0, RES - 1) for j in range(3))

                for i in range(3):
                    for j in range(3):
                        flat = nxs[i] + nys[j]
                        ww = wx[i] * wy[j]
                        mass_c = wxm[i] * wy[j]
                        momx_c = ww * (xu[i] + v_j[j])
                        momy_c = ww * (yv[i] + s_j[j])
                        if half == 0:
                            idx = flat
                            m = flat < HALF
                        else:
                            idx = flat - HALF
                            m = flat >= HALF
                        plsc.addupdate_scatter(mass_v, [idx], mass_c, mask=m)
                        plsc.addupdate_scatter(mx_v, [idx], momx_c, mask=m)
                        plsc.addupdate_scatter(my_v, [idx], momy_c, mask=m)
                return gcarry

            lax.fori_loop(0, GROUPS, group_body, 0)
            return carry

        lax.fori_loop(0, N_CHUNK, chunk_body, 0)

        pltpu.sync_copy(mass_v, out_hbm.at[3 * half + 0, wid, 0])
        pltpu.sync_copy(mx_v, out_hbm.at[3 * half + 1, wid, 0])
        pltpu.sync_copy(my_v, out_hbm.at[3 * half + 2, wid, 0])


def _red_body(x_ref, o_ref):
    o_ref[...] = jnp.sum(x_ref[...], axis=1)


_reduce_tc = pl.pallas_call(
    _red_body,
    grid=(6,),
    in_specs=[pl.BlockSpec((1, N_TILES, 1, HALF), lambda k: (k, 0, 0, 0))],
    out_specs=pl.BlockSpec((1, 1, HALF), lambda k: (k, 0, 0)),
    out_shape=jax.ShapeDtypeStruct((6, 1, HALF), jnp.float32),
)


def kernel(pos, vel, F, C, Jp):
    del Jp
    parts = _p2g_sc(
        pos[:, 0], pos[:, 1], vel[:, 0], vel[:, 1],
        F[:, 0, 0], F[:, 0, 1], F[:, 1, 0], F[:, 1, 1],
        C[:, 0, 0], C[:, 0, 1], C[:, 1, 0], C[:, 1, 1])
    red = _reduce_tc(parts).reshape(2, 3, HALF)
    grid_mass = jnp.concatenate([red[0, 0], red[1, 0]])
    momx = jnp.concatenate([red[0, 1], red[1, 1]])
    momy = jnp.concatenate([red[0, 2], red[1, 2]])
    grid_mom = jnp.stack([momx, momy], axis=-1)
    return grid_mom, grid_mass
